# Initial kernel scaffold; baseline (speedup 1.0000x reference)
import jax
import jax.numpy as jnp
from jax.experimental import pallas as pl


def kernel(static_numeric, station_index, district_id, operation_type_id, station_emb, district_emb, op_emb, W1, b1, W2, b2):
    raise NotImplementedError("write your pallas kernel here")


# speedup vs baseline: 2.6690x; 2.6690x over previous
import functools

import jax
import jax.numpy as jnp
from jax import lax
from jax.experimental import pallas as pl
from jax.experimental.pallas import tpu as pltpu
from jax.experimental.pallas import tpu_sc as plsc

B, S = 4096, 200
N = B * S
NUM_STATIONS, NUM_DISTRICTS, NUM_OPS = 100000, 1000, 8
CTX = 32

NC, NS = 2, 16
NW = NC * NS
IDX_ROWS = N // 128
ROWS_PER_W = IDX_ROWS // NW
CHUNK_ROWS = 8
CHUNK_TOK = CHUNK_ROWS * 128
N_CHUNKS = ROWS_PER_W // CHUNK_ROWS


def _prep_body(se_ref, w1a_ref, de_ref, opv_ref, w1b_ref, w1c_ref,
               tst_ref, tdo_ref):
    tst_ref[...] = jnp.dot(se_ref[...], w1a_ref[...],
                           preferred_element_type=jnp.float32
                           ).astype(jnp.bfloat16)
    do = (jnp.dot(de_ref[...], w1b_ref[...],
                  preferred_element_type=jnp.float32)
          + opv_ref[...] * w1c_ref[...])
    tdo_ref[...] = do.astype(jnp.bfloat16)


PREP_G = 10
ST_BLK = NUM_STATIONS // PREP_G
DO_BLK = NUM_DISTRICTS * NUM_OPS // PREP_G


def _prep_tables(station_emb, w1a, de_rep, op_rep, w1b, w1c):
    return pl.pallas_call(
        _prep_body,
        grid=(PREP_G,),
        in_specs=[
            pl.BlockSpec((ST_BLK, 16), lambda i: (i, 0)),
            pl.BlockSpec((16, CTX), lambda i: (0, 0)),
            pl.BlockSpec((DO_BLK, 8), lambda i: (i, 0)),
            pl.BlockSpec((DO_BLK, 1), lambda i: (i, 0)),
            pl.BlockSpec((8, CTX), lambda i: (0, 0)),
            pl.BlockSpec((1, CTX), lambda i: (0, 0)),
        ],
        out_specs=(
            pl.BlockSpec((ST_BLK, CTX), lambda i: (i, 0)),
            pl.BlockSpec((DO_BLK, CTX), lambda i: (i, 0)),
        ),
        out_shape=(
            jax.ShapeDtypeStruct((NUM_STATIONS, CTX), jnp.bfloat16),
            jax.ShapeDtypeStruct((NUM_DISTRICTS * NUM_OPS, CTX), jnp.bfloat16),
        ),
    )(station_emb, w1a, de_rep, op_rep, w1b, w1c)


PACK = CTX // 2


def _sc_gather_body(st_idx, di_idx, op_idx, t_st, t_do,
                    g_st, g_do,
                    ist_v, idi_v, iop_v, ido_v, bst_v, bdo_v, sem):
    wid = lax.axis_index("s") * NC + lax.axis_index("c")
    base_row = wid * ROWS_PER_W

    def chunk_body(ci, _):
        row0 = base_row + ci * CHUNK_ROWS
        tok0 = row0 * 128
        pltpu.sync_copy(st_idx.at[pl.ds(row0, CHUNK_ROWS)], ist_v)
        pltpu.sync_copy(di_idx.at[pl.ds(row0, CHUNK_ROWS)], idi_v)
        pltpu.sync_copy(op_idx.at[pl.ds(row0, CHUNK_ROWS)], iop_v)
        for j in range(CHUNK_ROWS):
            for v in range(8):
                sl = pl.ds(v * 16, 16)
                d = idi_v[j, sl]
                o = jnp.maximum(iop_v[j, sl], 0)
                ido_v[j, sl] = d * NUM_OPS + o
        copies = []
        for j in range(CHUNK_ROWS):
            dst = pl.ds(j * 128, 128)
            copies.append(pltpu.async_copy(
                t_st.at[ist_v.at[j]], bst_v.at[dst], sem))
            copies.append(pltpu.async_copy(
                t_do.at[ido_v.at[j]], bdo_v.at[dst], sem))
        for c in copies:
            c.wait()
        pltpu.sync_copy(bst_v, g_st.at[pl.ds(tok0, CHUNK_TOK)])
        pltpu.sync_copy(bdo_v, g_do.at[pl.ds(tok0, CHUNK_TOK)])
        return _

    lax.fori_loop(0, N_CHUNKS, chunk_body, None)


def _sc_gather(st_idx, di_idx, op_idx, t_st, t_do):
    mesh = plsc.VectorSubcoreMesh(core_axis_name="c", subcore_axis_name="s")
    kern = functools.partial(
        pl.kernel,
        mesh=mesh,
        compiler_params=pltpu.CompilerParams(use_tc_tiling_on_sc=False),
        out_type=(
            jax.ShapeDtypeStruct((N, PACK), jnp.int32),
            jax.ShapeDtypeStruct((N, PACK), jnp.int32),
        ),
        scratch_types=[
            pltpu.VMEM((CHUNK_ROWS, 128), jnp.int32),
            pltpu.VMEM((CHUNK_ROWS, 128), jnp.int32),
            pltpu.VMEM((CHUNK_ROWS, 128), jnp.int32),
            pltpu.VMEM((CHUNK_ROWS, 128), jnp.int32),
            pltpu.VMEM((CHUNK_TOK, PACK), jnp.int32),
            pltpu.VMEM((CHUNK_TOK, PACK), jnp.int32),
            pltpu.SemaphoreType.DMA,
        ],
    )(_sc_gather_body)
    return kern(st_idx, di_idx, op_idx, t_st, t_do)


BB = 8
TB = BB * S


def _unpack_pair(w):
    lo = jax.lax.bitcast_convert_type(
        jax.lax.shift_left(w, jnp.int32(16)), jnp.float32)
    hi = jax.lax.bitcast_convert_type(
        jnp.bitwise_and(w, jnp.int32(-65536)), jnp.float32)
    return lo, hi


def _mlp_body(gst_ref, gdo_ref, stat_ref, w1d_ref, b1_ref, w2_ref, b2_ref,
              out_ref):
    st_lo, st_hi = _unpack_pair(gst_ref[...])
    do_lo, do_hi = _unpack_pair(gdo_ref[...])
    g = jnp.concatenate([st_lo + do_lo, st_hi + do_hi], axis=1)
    stat = stat_ref[...].reshape(TB, 16)
    acc = g + jnp.dot(stat, w1d_ref[...],
                      preferred_element_type=jnp.float32) + b1_ref[...]
    h = jnp.maximum(acc, 0.0)
    res = jnp.dot(h, w2_ref[...],
                  preferred_element_type=jnp.float32) + b2_ref[...]
    out_ref[...] = res.reshape(BB, S, CTX)


def _mlp(g_st, g_do, static3d, w1d, b1, w2, b2):
    grid = (B // BB,)
    return pl.pallas_call(
        _mlp_body,
        grid=grid,
        in_specs=[
            pl.BlockSpec((TB, PACK), lambda i: (i, 0)),
            pl.BlockSpec((TB, PACK), lambda i: (i, 0)),
            pl.BlockSpec((BB, S, 16), lambda i: (i, 0, 0)),
            pl.BlockSpec((16, CTX), lambda i: (0, 0)),
            pl.BlockSpec((1, CTX), lambda i: (0, 0)),
            pl.BlockSpec((CTX, CTX), lambda i: (0, 0)),
            pl.BlockSpec((1, CTX), lambda i: (0, 0)),
        ],
        out_specs=pl.BlockSpec((BB, S, CTX), lambda i: (i, 0, 0)),
        out_shape=jax.ShapeDtypeStruct((B, S, CTX), jnp.float32),
    )(g_st, g_do, static3d, w1d, b1, w2, b2)


def kernel(static_numeric, station_index, district_id, operation_type_id,
           station_emb, district_emb, op_emb, W1, b1, W2, b2):
    st_idx = station_index.astype(jnp.int32).reshape(IDX_ROWS, 128)
    di_idx = district_id.astype(jnp.int32).reshape(IDX_ROWS, 128)
    op_idx = operation_type_id.astype(jnp.int32).reshape(IDX_ROWS, 128)

    q = jnp.arange(CTX).reshape(2, PACK).T.reshape(CTX)
    w1a = W1[0:16, q]
    w1b = W1[16:24, q]
    w1c = W1[24:25, q]
    w1d = W1[25:41, :]
    de_rep = jnp.repeat(district_emb, NUM_OPS, axis=0)
    op_rep = jnp.tile(op_emb, (NUM_DISTRICTS, 1))

    t_st_b, t_do_b = _prep_tables(station_emb, w1a, de_rep, op_rep, w1b, w1c)
    t_st = jax.lax.bitcast_convert_type(
        t_st_b.reshape(NUM_STATIONS, PACK, 2), jnp.int32)
    t_do = jax.lax.bitcast_convert_type(
        t_do_b.reshape(NUM_DISTRICTS * NUM_OPS, PACK, 2), jnp.int32)
    g_st, g_do = _sc_gather(st_idx, di_idx, op_idx, t_st, t_do)
    return _mlp(g_st, g_do, static_numeric.astype(jnp.float32), w1d,
                b1.reshape(1, CTX), W2, b2.reshape(1, CTX))


# speedup vs baseline: 5.5070x; 2.0633x over previous
import functools

import jax
import jax.numpy as jnp
from jax import lax
from jax.experimental import pallas as pl
from jax.experimental.pallas import tpu as pltpu
from jax.experimental.pallas import tpu_sc as plsc

B, S = 4096, 200
N = B * S
NUM_STATIONS, NUM_DISTRICTS, NUM_OPS = 100000, 1000, 8
NUM_DO = NUM_DISTRICTS * NUM_OPS
CTX = 32
PACK = CTX // 2

NC, NS = 2, 16
NW = NC * NS
IDX_ROWS = N // 128
ROWS_PER_W = IDX_ROWS // NW
CHUNK_ROWS = 8
CHUNK_TOK = CHUNK_ROWS * 128
N_CHUNKS = ROWS_PER_W // CHUNK_ROWS


def _pack_pair(x):
    b = jax.lax.bitcast_convert_type(x.astype(jnp.bfloat16), jnp.uint16)
    w = b.astype(jnp.uint32)
    lo = w[:PACK, :]
    hi = w[PACK:, :]
    return jax.lax.bitcast_convert_type(
        lo | (hi << jnp.uint32(16)), jnp.int32)


def _prep_body(se_ref, w1a_ref, de_ref, opv_ref, w1b_ref, w1c_ref,
               tst_ref, tdo_ref):
    st = jnp.dot(w1a_ref[...], se_ref[...],
                 preferred_element_type=jnp.float32)
    tst_ref[...] = _pack_pair(st)
    do = (jnp.dot(w1b_ref[...], de_ref[...],
                  preferred_element_type=jnp.float32)
          + w1c_ref[...] * opv_ref[...])
    tdo_ref[...] = _pack_pair(do)


NUM_ST_PAD = 102400
NUM_DO_PAD = 8192
PREP_G = 8
ST_BLK = NUM_ST_PAD // PREP_G
DO_BLK = NUM_DO_PAD // PREP_G


def _prep_tables(se_t, w1a_t, de_rep_t, op_rep_t, w1b_t, w1c_t):
    return pl.pallas_call(
        _prep_body,
        grid=(PREP_G,),
        in_specs=[
            pl.BlockSpec((16, ST_BLK), lambda i: (0, i)),
            pl.BlockSpec((CTX, 16), lambda i: (0, 0)),
            pl.BlockSpec((8, DO_BLK), lambda i: (0, i)),
            pl.BlockSpec((1, DO_BLK), lambda i: (0, i)),
            pl.BlockSpec((CTX, 8), lambda i: (0, 0)),
            pl.BlockSpec((CTX, 1), lambda i: (0, 0)),
        ],
        out_specs=(
            pl.BlockSpec((PACK, ST_BLK), lambda i: (0, i)),
            pl.BlockSpec((PACK, DO_BLK), lambda i: (0, i)),
        ),
        out_shape=(
            jax.ShapeDtypeStruct((PACK, NUM_ST_PAD), jnp.int32),
            jax.ShapeDtypeStruct((PACK, NUM_DO_PAD), jnp.int32),
        ),
    )(se_t, w1a_t, de_rep_t, op_rep_t, w1b_t, w1c_t)


def _sc_gather_body(st_idx, di_idx, op_idx, t_st, t_do,
                    g_st, g_do,
                    ist_v, idi_v, iop_v, ido_v, bst_v, bdo_v, sem):
    wid = lax.axis_index("s") * NC + lax.axis_index("c")
    base_row = wid * ROWS_PER_W

    def chunk_body(ci, _):
        row0 = base_row + ci * CHUNK_ROWS
        tok0 = row0 * 128
        pltpu.sync_copy(st_idx.at[pl.ds(row0, CHUNK_ROWS)], ist_v)
        pltpu.sync_copy(di_idx.at[pl.ds(row0, CHUNK_ROWS)], idi_v)
        pltpu.sync_copy(op_idx.at[pl.ds(row0, CHUNK_ROWS)], iop_v)
        for j in range(CHUNK_ROWS):
            for v in range(8):
                sl = pl.ds(v * 16, 16)
                d = idi_v[j, sl]
                o = jnp.maximum(iop_v[j, sl], 0)
                ido_v[j, sl] = d * NUM_OPS + o
        copies = []
        for j in range(CHUNK_ROWS):
            dst = pl.ds(j * 128, 128)
            copies.append(pltpu.async_copy(
                t_st.at[ist_v.at[j]], bst_v.at[dst], sem))
            copies.append(pltpu.async_copy(
                t_do.at[ido_v.at[j]], bdo_v.at[dst], sem))
        for c in copies:
            c.wait()
        pltpu.sync_copy(bst_v, g_st.at[pl.ds(tok0, CHUNK_TOK)])
        pltpu.sync_copy(bdo_v, g_do.at[pl.ds(tok0, CHUNK_TOK)])
        return _

    lax.fori_loop(0, N_CHUNKS, chunk_body, None)


def _sc_gather(st_idx, di_idx, op_idx, t_st, t_do):
    mesh = plsc.VectorSubcoreMesh(core_axis_name="c", subcore_axis_name="s")
    kern = functools.partial(
        pl.kernel,
        mesh=mesh,
        compiler_params=pltpu.CompilerParams(use_tc_tiling_on_sc=False),
        out_type=(
            jax.ShapeDtypeStruct((N, PACK), jnp.int32),
            jax.ShapeDtypeStruct((N, PACK), jnp.int32),
        ),
        scratch_types=[
            pltpu.VMEM((CHUNK_ROWS, 128), jnp.int32),
            pltpu.VMEM((CHUNK_ROWS, 128), jnp.int32),
            pltpu.VMEM((CHUNK_ROWS, 128), jnp.int32),
            pltpu.VMEM((CHUNK_ROWS, 128), jnp.int32),
            pltpu.VMEM((CHUNK_TOK, PACK), jnp.int32),
            pltpu.VMEM((CHUNK_TOK, PACK), jnp.int32),
            pltpu.SemaphoreType.DMA,
        ],
    )(_sc_gather_body)
    return kern(st_idx, di_idx, op_idx, t_st, t_do)


GROWS = B * PACK // 128


def _unpack_pair(w):
    lo = jax.lax.bitcast_convert_type(
        jax.lax.shift_left(w, jnp.int32(16)), jnp.float32)
    hi = jax.lax.bitcast_convert_type(
        jnp.bitwise_and(w, jnp.int32(-65536)), jnp.float32)
    return lo, hi


def _mlp_body(gst_ref, gdo_ref, stat_ref, w1d_ref, b1_ref, w2_ref, b2_ref,
              out_ref):
    st_lo, st_hi = _unpack_pair(gst_ref[0])
    do_lo, do_hi = _unpack_pair(gdo_ref[0])
    g = jnp.concatenate([st_lo + do_lo, st_hi + do_hi], axis=0)
    acc = jnp.dot(w1d_ref[...], stat_ref[0],
                  preferred_element_type=jnp.float32) + g + b1_ref[...]
    h = jnp.maximum(acc, 0.0)
    out_ref[0] = jnp.dot(w2_ref[...], h,
                         preferred_element_type=jnp.float32) + b2_ref[...]


def _mlp(g_st3, g_do3, static_t, w1d_t, b1c, w2_t, b2c):
    return pl.pallas_call(
        _mlp_body,
        grid=(S,),
        in_specs=[
            pl.BlockSpec((1, PACK, B), lambda i: (i, 0, 0)),
            pl.BlockSpec((1, PACK, B), lambda i: (i, 0, 0)),
            pl.BlockSpec((1, 16, B), lambda i: (i, 0, 0)),
            pl.BlockSpec((CTX, 16), lambda i: (0, 0)),
            pl.BlockSpec((CTX, 1), lambda i: (0, 0)),
            pl.BlockSpec((CTX, CTX), lambda i: (0, 0)),
            pl.BlockSpec((CTX, 1), lambda i: (0, 0)),
        ],
        out_specs=pl.BlockSpec((1, CTX, B), lambda i: (i, 0, 0)),
        out_shape=jax.ShapeDtypeStruct((S, CTX, B), jnp.float32),
    )(g_st3, g_do3, static_t, w1d_t, b1c, w2_t, b2c)


def kernel(static_numeric, station_index, district_id, operation_type_id,
           station_emb, district_emb, op_emb, W1, b1, W2, b2):
    st_idx = station_index.astype(jnp.int32).T.reshape(IDX_ROWS, 128)
    di_idx = district_id.astype(jnp.int32).T.reshape(IDX_ROWS, 128)
    op_idx = operation_type_id.astype(jnp.int32).T.reshape(IDX_ROWS, 128)
    static_t = static_numeric.astype(jnp.float32).transpose(1, 2, 0)

    w1a_t = W1[0:16, :].T
    w1b_t = W1[16:24, :].T
    w1c_t = W1[24:25, :].T
    w1d_t = W1[25:41, :].T
    de_rep_t = jnp.repeat(district_emb.T, NUM_OPS, axis=1)
    op_rep_t = jnp.tile(op_emb.T, (1, NUM_DISTRICTS))

    se_t = jnp.pad(station_emb.T, ((0, 0), (0, NUM_ST_PAD - NUM_STATIONS)))
    de_rep_t = jnp.pad(de_rep_t, ((0, 0), (0, NUM_DO_PAD - NUM_DO)))
    op_rep_t = jnp.pad(op_rep_t, ((0, 0), (0, NUM_DO_PAD - NUM_DO)))

    tst_t, tdo_t = _prep_tables(
        se_t, w1a_t, de_rep_t, op_rep_t, w1b_t, w1c_t)
    t_st = tst_t.T
    t_do = tdo_t.T

    g_st, g_do = _sc_gather(st_idx, di_idx, op_idx, t_st, t_do)
    g_st3 = g_st.reshape(S, B, PACK).transpose(0, 2, 1)
    g_do3 = g_do.reshape(S, B, PACK).transpose(0, 2, 1)

    out_t = _mlp(g_st3, g_do3, static_t, w1d_t, b1.reshape(CTX, 1), W2.T,
                 b2.reshape(CTX, 1))
    return out_t.transpose(2, 0, 1)
